# baseline (speedup 1.0000x reference)
import functools

import jax
import jax.numpy as jnp
from jax import lax
from jax.experimental import pallas as pl
from jax.experimental.pallas import tpu as pltpu
from jax.experimental.pallas import tpu_sc as plsc

N = 4096
C = 100000
L = 16
NC = 2
NS = 16
NW = NC * NS
BW = N // NW
CH = 32
NCHK = BW // CH
NG = BW // L


def _sc_body(prob_hbm, tgt_hbm, rew_hbm, part_hbm, out_hbm,
             tgt_v, rew_v, vals_v, acc_v, all_v, sem0, sem1):
    cid = lax.axis_index("c")
    sid = lax.axis_index("s")
    wid = cid * NS + sid
    base = wid * BW

    pltpu.sync_copy(tgt_hbm.at[pl.ds(base, BW)], tgt_v)
    pltpu.sync_copy(rew_hbm.at[pl.ds(base, BW)], rew_v)

    lane = lax.iota(jnp.int32, L)
    sems = [sem0, sem1]

    tgts = [tgt_v[pl.ds(q * L, L)] for q in range(NG)]
    colt = [lax.mul(lax.div(t, 128), 128) for t in tgts]
    sub16 = [lax.mul(lax.div(lax.rem(t, 128), L), L) for t in tgts]
    offs = [lax.rem(t, L) for t in tgts]
    rews = [rew_v[pl.ds(q * L, L)] for q in range(NG)]

    def fire(c):
        buf = c % 2
        cps = []
        for j in range(CH):
            e = c * CH + j
            q, i = e // L, e % L
            row8 = pl.multiple_of(base + (e // 8) * 8, 8)
            col = pl.multiple_of(colt[q][i], 128)
            cps.append(pltpu.async_copy(
                prob_hbm.at[pl.ds(row8, 8), pl.ds(col, 128)],
                vals_v.at[buf, j], sems[buf]))
        return cps

    def compute(c, acc):
        buf = c % 2
        for j in range(CH):
            e = c * CH + j
            q, i = e // L, e % L
            row = vals_v[buf, j, e % 8, pl.ds(sub16[q][i], L)]
            sel = jnp.where(lane == offs[q][i], rews[q][i], 0.0)
            acc = acc + row * sel
        return acc

    acc = jnp.zeros((L,), jnp.float32)
    inflight = fire(0)
    for c in range(NCHK):
        nxt = fire(c + 1) if c + 1 < NCHK else None
        for cp in inflight:
            cp.wait()
        acc = compute(c, acc)
        inflight = nxt
    acc_v[...] = acc

    pltpu.sync_copy(acc_v, part_hbm.at[wid])
    plsc.subcore_barrier()

    @pl.when(sid == 0)
    def _():
        pltpu.sync_copy(part_hbm.at[pl.ds(cid * NS, NS)], all_v)
        tot = jnp.zeros((L,), jnp.float32)
        for r in range(NS):
            tot = tot + all_v[r]
        s = tot[0]
        for i in range(1, L):
            s = s + tot[i]
        acc_v[...] = lax.broadcast_in_dim(-s, (L,), ())
        pltpu.sync_copy(acc_v, out_hbm.at[cid])


@jax.jit
def _sc_loss(prob, target, reward):
    mesh = plsc.VectorSubcoreMesh(core_axis_name="c", subcore_axis_name="s")
    f = functools.partial(
        pl.kernel,
        out_type=(jax.ShapeDtypeStruct((NW, L), jnp.float32),
                  jax.ShapeDtypeStruct((NC, L), jnp.float32)),
        mesh=mesh,
        compiler_params=pltpu.CompilerParams(skip_device_barrier=True),
        scratch_types=[
            pltpu.VMEM((BW,), jnp.int32),
            pltpu.VMEM((BW,), jnp.float32),
            pltpu.VMEM((2, CH, 8, 128), jnp.float32),
            pltpu.VMEM((L,), jnp.float32),
            pltpu.VMEM((NS, L), jnp.float32),
            pltpu.SemaphoreType.DMA,
            pltpu.SemaphoreType.DMA,
        ],
    )(_sc_body)
    return f(prob, target, reward)


def kernel(prob, target, reward):
    _, out = _sc_loss(prob, target, reward)
    return out[0, 0] + out[1, 0]


# speedup vs baseline: 53.7612x; 53.7612x over previous
import functools

import jax
import jax.numpy as jnp
from jax import lax
from jax.experimental import pallas as pl
from jax.experimental.pallas import tpu as pltpu
from jax.experimental.pallas import tpu_sc as plsc

N = 4096
C = 100000
L = 16
NC = 2
NS = 16
NW = NC * NS
BW = N // NW
NG = BW // L
NROWS = N * C // 128


def _sc_body(phy_hbm, tgt_hbm, rew_hbm, part_hbm, out_hbm,
             tgt_v, rew_v, idx_v, vals_v, acc_v, all_v, sem):
    cid = lax.axis_index("c")
    sid = lax.axis_index("s")
    wid = cid * NS + sid
    base = wid * BW

    pltpu.sync_copy(tgt_hbm.at[pl.ds(base, BW)], tgt_v)
    pltpu.sync_copy(rew_hbm.at[pl.ds(base, BW)], rew_v)

    lane = lax.iota(jnp.int32, L)

    for q in range(NG):
        t = tgt_v[pl.ds(q * L, L)]
        idx_v[0, pl.ds(q * L, L)] = (
            lax.div(t, 8) * 256 + wid * 8 + lax.rem(t, 8))

    pltpu.async_copy(phy_hbm.at[idx_v.at[0]], vals_v, sem).wait()

    acc = jnp.zeros((L,), jnp.float32)
    for q in range(NG):
        rews = rew_v[pl.ds(q * L, L)]
        for i in range(L):
            j = q * L + i
            row16 = vals_v[j, pl.ds(q * L, L)]
            sel = jnp.where(lane == i, rews[i], 0.0)
            acc = acc + row16 * sel
    acc_v[...] = acc

    pltpu.sync_copy(acc_v, part_hbm.at[wid])
    plsc.subcore_barrier()

    @pl.when(sid == 0)
    def _():
        pltpu.sync_copy(part_hbm.at[pl.ds(cid * NS, NS)], all_v)
        tot = jnp.zeros((L,), jnp.float32)
        for r in range(NS):
            tot = tot + all_v[r]
        s = tot[0]
        for i in range(1, L):
            s = s + tot[i]
        acc_v[...] = lax.broadcast_in_dim(-s, (L,), ())
        pltpu.sync_copy(acc_v, out_hbm.at[cid])


@jax.jit
def _sc_loss(phy, target, reward):
    mesh = plsc.VectorSubcoreMesh(core_axis_name="c", subcore_axis_name="s")
    f = functools.partial(
        pl.kernel,
        out_type=(jax.ShapeDtypeStruct((NW, L), jnp.float32),
                  jax.ShapeDtypeStruct((NC, L), jnp.float32)),
        mesh=mesh,
        compiler_params=pltpu.CompilerParams(skip_device_barrier=True),
        scratch_types=[
            pltpu.VMEM((BW,), jnp.int32),
            pltpu.VMEM((BW,), jnp.float32),
            pltpu.VMEM((1, BW), jnp.int32),
            pltpu.VMEM((BW, 128), jnp.float32),
            pltpu.VMEM((L,), jnp.float32),
            pltpu.VMEM((NS, L), jnp.float32),
            pltpu.SemaphoreType.DMA,
        ],
    )(_sc_body)
    return f(phy, target, reward)


def kernel(prob, target, reward):
    phy = (prob.T.reshape(C // 8, 8, N // 128, 128)
           .transpose(0, 2, 1, 3).reshape(NROWS, 128))
    _, out = _sc_loss(phy, target, reward)
    return out[0, 0] + out[1, 0]


# speedup vs baseline: 54.0823x; 1.0060x over previous
import functools

import jax
import jax.numpy as jnp
from jax import lax
from jax.experimental import pallas as pl
from jax.experimental.pallas import tpu as pltpu
from jax.experimental.pallas import tpu_sc as plsc

N = 4096
C = 100000
L = 16
NC = 2
NS = 16
NW = NC * NS
BW = N // NW
NG = BW // L
NROWS = N * C // 128


def _sc_body(phy_hbm, tgt_hbm, rew_hbm, part_hbm, out_hbm,
             tgt_v, rew_v, idx_v, vals_v, acc_v, all_v, sem):
    cid = lax.axis_index("c")
    sid = lax.axis_index("s")
    wid = cid * NS + sid
    base = wid * BW

    pltpu.sync_copy(tgt_hbm.at[pl.ds(base, BW)], tgt_v)
    pltpu.sync_copy(rew_hbm.at[pl.ds(base, BW)], rew_v)

    lane = lax.iota(jnp.int32, L)

    for q in range(NG):
        t = tgt_v[pl.ds(q * L, L)]
        idx_v[q // 4, pl.ds((q % 4) * L, L)] = (
            lax.div(t, 8) * 256 + wid * 8 + lax.rem(t, 8))

    cp0 = pltpu.async_copy(phy_hbm.at[idx_v.at[0]], vals_v.at[0], sem)
    cp1 = pltpu.async_copy(phy_hbm.at[idx_v.at[1]], vals_v.at[1], sem)

    acc = jnp.zeros((L,), jnp.float32)
    cp0.wait()
    for h in range(2):
        if h == 1:
            cp1.wait()
        for q in range(NG // 2):
            g = h * (NG // 2) + q
            rews = rew_v[pl.ds(g * L, L)]
            for i in range(L):
                row16 = vals_v[h, q * L + i, pl.ds(g * L, L)]
                sel = jnp.where(lane == i, rews[i], 0.0)
                acc = acc + row16 * sel
    acc_v[...] = acc

    pltpu.sync_copy(acc_v, part_hbm.at[wid])
    plsc.subcore_barrier()

    @pl.when(sid == 0)
    def _():
        pltpu.sync_copy(part_hbm.at[pl.ds(cid * NS, NS)], all_v)
        tot = jnp.zeros((L,), jnp.float32)
        for r in range(NS):
            tot = tot + all_v[r]
        s = tot[0]
        for i in range(1, L):
            s = s + tot[i]
        acc_v[...] = lax.broadcast_in_dim(-s, (L,), ())
        pltpu.sync_copy(acc_v, out_hbm.at[cid])


@jax.jit
def _sc_loss(phy, target, reward):
    mesh = plsc.VectorSubcoreMesh(core_axis_name="c", subcore_axis_name="s")
    f = functools.partial(
        pl.kernel,
        out_type=(jax.ShapeDtypeStruct((NW, L), jnp.float32),
                  jax.ShapeDtypeStruct((NC, L), jnp.float32)),
        mesh=mesh,
        compiler_params=pltpu.CompilerParams(skip_device_barrier=True),
        scratch_types=[
            pltpu.VMEM((BW,), jnp.int32),
            pltpu.VMEM((BW,), jnp.float32),
            pltpu.VMEM((2, BW // 2), jnp.int32),
            pltpu.VMEM((2, BW // 2, 128), jnp.float32),
            pltpu.VMEM((L,), jnp.float32),
            pltpu.VMEM((NS, L), jnp.float32),
            pltpu.SemaphoreType.DMA,
        ],
    )(_sc_body)
    return f(phy, target, reward)


def kernel(prob, target, reward):
    phy = (prob.T.reshape(C // 8, 8, N // 128, 128)
           .transpose(0, 2, 1, 3).reshape(NROWS, 128))
    _, out = _sc_loss(phy, target, reward)
    return out[0, 0] + out[1, 0]


# speedup vs baseline: 60.9044x; 1.1261x over previous
import functools

import jax
import jax.numpy as jnp
from jax import lax
from jax.experimental import pallas as pl
from jax.experimental.pallas import tpu as pltpu
from jax.experimental.pallas import tpu_sc as plsc

N = 4096
C = 100000
L = 16
NC = 2
NS = 16
NW = NC * NS
BW = N // NW
NG = BW // L
NROWS = N * C // 128


def _sc_body(phy_hbm, tgt_hbm, rew_hbm, part_hbm,
             tgt_v, rew_v, idx_v, vals_v, acc_v, sem_in, sem):
    cid = lax.axis_index("c")
    sid = lax.axis_index("s")
    wid = cid * NS + sid
    base = wid * BW

    cp_t = pltpu.async_copy(tgt_hbm.at[pl.ds(base, BW)], tgt_v, sem_in)
    cp_r = pltpu.async_copy(rew_hbm.at[pl.ds(base, BW)], rew_v, sem_in)
    cp_t.wait()

    lane = lax.iota(jnp.int32, L)

    for q in range(NG):
        t = tgt_v[pl.ds(q * L, L)]
        idx_v[q // 4, pl.ds((q % 4) * L, L)] = (
            lax.div(t, 8) * 256 + wid * 8 + lax.rem(t, 8))

    cp0 = pltpu.async_copy(phy_hbm.at[idx_v.at[0]], vals_v.at[0], sem)
    cp1 = pltpu.async_copy(phy_hbm.at[idx_v.at[1]], vals_v.at[1], sem)
    cp_r.wait()

    acc = jnp.zeros((L,), jnp.float32)
    cp0.wait()
    for h in range(2):
        if h == 1:
            cp1.wait()
        for q in range(NG // 2):
            g = h * (NG // 2) + q
            rews = rew_v[pl.ds(g * L, L)]
            for i in range(L):
                row16 = vals_v[h, q * L + i, pl.ds(g * L, L)]
                sel = jnp.where(lane == i, rews[i], 0.0)
                acc = acc - row16 * sel
    acc_v[...] = acc
    pltpu.sync_copy(acc_v, part_hbm.at[wid])


@jax.jit
def _sc_loss(phy, target, reward):
    mesh = plsc.VectorSubcoreMesh(core_axis_name="c", subcore_axis_name="s")
    f = functools.partial(
        pl.kernel,
        out_type=jax.ShapeDtypeStruct((NW, L), jnp.float32),
        mesh=mesh,
        compiler_params=pltpu.CompilerParams(skip_device_barrier=True),
        scratch_types=[
            pltpu.VMEM((BW,), jnp.int32),
            pltpu.VMEM((BW,), jnp.float32),
            pltpu.VMEM((2, BW // 2), jnp.int32),
            pltpu.VMEM((2, BW // 2, 128), jnp.float32),
            pltpu.VMEM((L,), jnp.float32),
            pltpu.SemaphoreType.DMA,
            pltpu.SemaphoreType.DMA,
        ],
    )(_sc_body)
    return f(phy, target, reward)


def kernel(prob, target, reward):
    phy = (prob.T.reshape(C // 8, 8, N // 128, 128)
           .transpose(0, 2, 1, 3).reshape(NROWS, 128))
    part = _sc_loss(phy, target, reward)
    return jnp.sum(part)


# speedup vs baseline: 64.2161x; 1.0544x over previous
import functools

import jax
import jax.numpy as jnp
from jax import lax
from jax.experimental import pallas as pl
from jax.experimental.pallas import tpu as pltpu
from jax.experimental.pallas import tpu_sc as plsc

N = 4096
C = 100000
L = 16
NC = 2
NS = 16
NW = NC * NS
BW = N // NW
NG = BW // L
NROWS = N * C // 128


def _sc_body(phy_hbm, tgt_hbm, rew_hbm, part_hbm,
             tgt_v, rew_v, idx_v, vals_v, acc_v, sem_in, sem):
    cid = lax.axis_index("c")
    sid = lax.axis_index("s")
    wid = cid * NS + sid
    base = wid * BW

    cp_t = pltpu.async_copy(tgt_hbm.at[pl.ds(base, BW)], tgt_v, sem_in)
    cp_r = pltpu.async_copy(rew_hbm.at[pl.ds(base, BW)], rew_v, sem_in)
    cp_t.wait()

    lane = lax.iota(jnp.int32, L)

    for q in range(NG):
        t = tgt_v[pl.ds(q * L, L)]
        idx_v[0, pl.ds(q * L, L)] = (
            lax.div(t, 8) * 32768 + wid * 1024 + lax.rem(t, 8) * 128
            + q * L + lane)

    cp0 = pltpu.async_copy(phy_hbm.at[idx_v.at[0]], vals_v.at[0], sem)
    cp_r.wait()
    cp0.wait()

    acc = jnp.zeros((L,), jnp.float32)
    for q in range(NG):
        acc = acc - vals_v[0, pl.ds(q * L, L)] * rew_v[pl.ds(q * L, L)]
    acc_v[...] = acc
    pltpu.sync_copy(acc_v, part_hbm.at[wid])


@jax.jit
def _sc_loss(phy, target, reward):
    mesh = plsc.VectorSubcoreMesh(core_axis_name="c", subcore_axis_name="s")
    f = functools.partial(
        pl.kernel,
        out_type=jax.ShapeDtypeStruct((NW, L), jnp.float32),
        mesh=mesh,
        compiler_params=pltpu.CompilerParams(skip_device_barrier=True),
        scratch_types=[
            pltpu.VMEM((BW,), jnp.int32),
            pltpu.VMEM((BW,), jnp.float32),
            pltpu.VMEM((1, BW), jnp.int32),
            pltpu.VMEM((1, BW), jnp.float32),
            pltpu.VMEM((L,), jnp.float32),
            pltpu.SemaphoreType.DMA,
            pltpu.SemaphoreType.DMA,
        ],
    )(_sc_body)
    return f(phy, target, reward)


def kernel(prob, target, reward):
    phy = (prob.T.reshape(C // 8, 8, N // 128, 128)
           .transpose(0, 2, 1, 3).reshape(N * C))
    part = _sc_loss(phy, target, reward)
    return jnp.sum(part)


# speedup vs baseline: 65.7000x; 1.0231x over previous
import functools

import jax
import jax.numpy as jnp
from jax import lax
from jax.experimental import pallas as pl
from jax.experimental.pallas import tpu as pltpu
from jax.experimental.pallas import tpu_sc as plsc

N = 4096
C = 100000
L = 16
NC = 2
NS = 16
NW = NC * NS
BW = N // NW
NG = BW // L
NROWS = N * C // 128


def _sc_body(phy_hbm, tgt_hbm, rew_hbm, part_hbm,
             tgt_v, rew_v, idx_v, vals_v, acc_v, sem_in, sem):
    cid = lax.axis_index("c")
    sid = lax.axis_index("s")
    wid = cid * NS + sid
    base = wid * BW

    cp_t = pltpu.async_copy(tgt_hbm.at[pl.ds(base, BW)], tgt_v, sem_in)
    cp_r = pltpu.async_copy(rew_hbm.at[pl.ds(base, BW)], rew_v, sem_in)
    cp_t.wait()

    lane = lax.iota(jnp.int32, L)

    for q in range(NG):
        t = tgt_v[pl.ds(q * L, L)]
        idx_v[q // 4, pl.ds((q % 4) * L, L)] = (
            lax.div(t, 8) * 32768 + wid * 1024 + lax.rem(t, 8) * 128
            + q * L + lane)

    cp0 = pltpu.async_copy(phy_hbm.at[idx_v.at[0]], vals_v.at[0], sem)
    cp1 = pltpu.async_copy(phy_hbm.at[idx_v.at[1]], vals_v.at[1], sem)
    cp_r.wait()

    acc = jnp.zeros((L,), jnp.float32)
    cp0.wait()
    for q in range(NG):
        if q == NG // 2:
            cp1.wait()
        acc = acc - (vals_v[q // 4, pl.ds((q % 4) * L, L)]
                     * rew_v[pl.ds(q * L, L)])
    acc_v[...] = acc
    pltpu.sync_copy(acc_v, part_hbm.at[wid])


@jax.jit
def _sc_loss(phy, target, reward):
    mesh = plsc.VectorSubcoreMesh(core_axis_name="c", subcore_axis_name="s")
    f = functools.partial(
        pl.kernel,
        out_type=jax.ShapeDtypeStruct((NW, L), jnp.float32),
        mesh=mesh,
        compiler_params=pltpu.CompilerParams(skip_device_barrier=True),
        scratch_types=[
            pltpu.VMEM((BW,), jnp.int32),
            pltpu.VMEM((BW,), jnp.float32),
            pltpu.VMEM((2, BW // 2), jnp.int32),
            pltpu.VMEM((2, BW // 2), jnp.float32),
            pltpu.VMEM((L,), jnp.float32),
            pltpu.SemaphoreType.DMA,
            pltpu.SemaphoreType.DMA,
        ],
    )(_sc_body)
    return f(phy, target, reward)


def kernel(prob, target, reward):
    phy = (prob.T.reshape(C // 8, 8, N // 128, 128)
           .transpose(0, 2, 1, 3).reshape(N * C))
    part = _sc_loss(phy, target, reward)
    return jnp.sum(part)
